# baseline (speedup 1.0000x reference)
import functools

import jax
import jax.numpy as jnp
from jax.experimental import pallas as pl
from jax.experimental.pallas import tpu as pltpu

SEQ_LEN = 2048
START_TOK = 0
END_TOK = 2
PAD_TOK = 1


def _trim_budgets(L1, L2, budget):
    if L1 + L2 <= budget:
        return L1, L2
    k1 = min(L1, max((budget + 1) // 2, budget - L2))
    k2 = min(L2, max(budget // 2, budget - L1))
    return max(k1, 0), max(k2, 0)


@functools.cache
def _build_packer(B, L1, L2):
    budget = SEQ_LEN - 4
    k1, k2 = _trim_budgets(L1, L2, budget)
    half = SEQ_LEN // 2
    assert k1 == half - 2 and k2 == half - 2
    hw = half // 2

    def body(s1_hbm, s2_hbm, o_hbm, a_v, b_v, o_v,
             sem_a0, sem_a1, sem_b0, sem_b1, sem_o):
        cpa0 = pltpu.make_async_copy(
            s1_hbm.at[:, pl.ds(0, hw)], a_v.at[:, pl.ds(0, hw)], sem_a0)
        cpa1 = pltpu.make_async_copy(
            s1_hbm.at[:, pl.ds(hw, hw)], a_v.at[:, pl.ds(hw, hw)], sem_a1)
        cpb0 = pltpu.make_async_copy(
            s2_hbm.at[:, pl.ds(0, hw)], b_v.at[:, pl.ds(0, hw)], sem_b0)
        cpb1 = pltpu.make_async_copy(
            s2_hbm.at[:, pl.ds(hw, hw)], b_v.at[:, pl.ds(hw, hw)], sem_b1)
        cpa0.start()
        cpa1.start()
        cpb0.start()
        cpb1.start()

        start = jnp.full((B, 1), START_TOK, jnp.int32)
        endc = jnp.full((B, 1), END_TOK, jnp.int32)

        outs = []

        def flush(c, block):
            o_v[:, pl.ds(c * hw, hw)] = block
            cp = pltpu.make_async_copy(
                o_v.at[:, pl.ds(c * hw, hw)],
                o_hbm.at[:, pl.ds(c * hw, hw)], sem_o)
            cp.start()
            outs.append(cp)

        cpa0.wait()
        flush(0, jnp.concatenate([start, a_v[:, :hw - 1]], axis=1))
        cpa1.wait()
        flush(1, jnp.concatenate(
            [a_v[:, hw - 1:k1], endc], axis=1))
        cpb0.wait()
        flush(2, jnp.concatenate([endc, b_v[:, :hw - 1]], axis=1))
        cpb1.wait()
        flush(3, jnp.concatenate(
            [b_v[:, hw - 1:k2], endc], axis=1))

        for cp in outs:
            cp.wait()

    return pl.pallas_call(
        body,
        in_specs=[
            pl.BlockSpec(memory_space=pltpu.MemorySpace.HBM),
            pl.BlockSpec(memory_space=pltpu.MemorySpace.HBM),
        ],
        out_specs=pl.BlockSpec(memory_space=pltpu.MemorySpace.HBM),
        out_shape=jax.ShapeDtypeStruct((B, SEQ_LEN), jnp.int32),
        scratch_shapes=[
            pltpu.VMEM((B, half), jnp.int32),
            pltpu.VMEM((B, half), jnp.int32),
            pltpu.VMEM((B, SEQ_LEN), jnp.int32),
            pltpu.SemaphoreType.DMA,
            pltpu.SemaphoreType.DMA,
            pltpu.SemaphoreType.DMA,
            pltpu.SemaphoreType.DMA,
            pltpu.SemaphoreType.DMA,
        ],
    )


def kernel(segment_1, segment_2):
    B, L1 = segment_1.shape
    L2 = segment_2.shape[1]
    return _build_packer(B, L1, L2)(segment_1, segment_2)
